# Initial kernel scaffold; baseline (speedup 1.0000x reference)
import functools

import jax
import jax.numpy as jnp
from jax import lax
from jax.experimental import pallas as pl
from jax.experimental.pallas import tpu as pltpu
from jax.experimental.pallas import tpu_sc as plsc

B = 16384
F = 26
NC, NS, L = 2, 16, 16
NW = NC * NS
RPW = B // NW
GPW = RPW * F


def _body(xf_hbm, table_hbm, bias_hbm, out_hbm, idx_v, rows_v, acc_v, bias_v, sem):
    wid = lax.axis_index("s") * NC + lax.axis_index("c")
    base = wid * RPW
    pltpu.sync_copy(xf_hbm.at[pl.ds(base * F, GPW)], idx_v)
    pltpu.sync_copy(bias_hbm, bias_v)
    pltpu.async_copy(table_hbm.at[idx_v], rows_v, sem).wait()

    lane = lax.iota(jnp.int32, L) * F

    def red(i, carry):
        b16 = pl.multiple_of(i * L, L)
        acc = bias_v[...]
        for f in range(F):
            acc = acc + plsc.load_gather(rows_v, [lane + (b16 * F + f)])
        acc_v[pl.ds(b16, L)] = acc
        return carry

    lax.fori_loop(0, RPW // L, red, 0)
    pltpu.sync_copy(acc_v, out_hbm.at[pl.ds(base, RPW)])


_launch = functools.partial(
    pl.kernel,
    out_type=jax.ShapeDtypeStruct((B,), jnp.float32),
    mesh=plsc.VectorSubcoreMesh(core_axis_name="c", subcore_axis_name="s"),
    scratch_types=[
        pltpu.VMEM((GPW,), jnp.int32),
        pltpu.VMEM((GPW,), jnp.float32),
        pltpu.VMEM((RPW,), jnp.float32),
        pltpu.VMEM((L,), jnp.float32),
        pltpu.SemaphoreType.DMA,
    ],
)(_body)


def kernel(X, table, bias):
    xf = X.reshape(-1)
    tf = table.reshape(-1)
    b16 = jnp.broadcast_to(bias, (L,))
    out = _launch(xf, tf, b16)
    return out.reshape(B, 1)


# speedup vs baseline: 1.4347x; 1.4347x over previous
import functools

import jax
import jax.numpy as jnp
from jax import lax
from jax.experimental import pallas as pl
from jax.experimental.pallas import tpu as pltpu
from jax.experimental.pallas import tpu_sc as plsc

B = 16384
F = 26
NC, NS, L = 2, 16, 16
NW = NC * NS
RPW = B // NW
GPW = RPW * F


def _body(xf_hbm, table_hbm, bias_hbm, out_hbm, idx_v, rows_v, acc_v, bias_v, sem):
    wid = lax.axis_index("s") * NC + lax.axis_index("c")
    base = wid * RPW
    pltpu.sync_copy(xf_hbm.at[pl.ds(wid * GPW, GPW)], idx_v)
    pltpu.sync_copy(bias_hbm, bias_v)
    pltpu.async_copy(table_hbm.at[idx_v], rows_v, sem).wait()

    def red(i, carry):
        b16 = pl.multiple_of(i * L, L)
        acc = bias_v[...]
        for f in range(F):
            acc = acc + rows_v[pl.ds(f * RPW + b16, L)]
        acc_v[pl.ds(b16, L)] = acc
        return carry

    lax.fori_loop(0, RPW // L, red, 0)
    pltpu.sync_copy(acc_v, out_hbm.at[pl.ds(base, RPW)])


_launch = functools.partial(
    pl.kernel,
    out_type=jax.ShapeDtypeStruct((B,), jnp.float32),
    mesh=plsc.VectorSubcoreMesh(
        core_axis_name="c", subcore_axis_name="s", num_cores=NC, num_subcores=NS),
    scratch_types=[
        pltpu.VMEM((GPW,), jnp.int32),
        pltpu.VMEM((GPW,), jnp.float32),
        pltpu.VMEM((RPW,), jnp.float32),
        pltpu.VMEM((L,), jnp.float32),
        pltpu.SemaphoreType.DMA,
    ],
)(_body)


def kernel(X, table, bias):
    xf = X.reshape(NW, RPW, F).transpose(0, 2, 1).reshape(-1)
    tf = table.reshape(-1)
    b16 = jnp.broadcast_to(bias, (L,))
    out = _launch(xf, tf, b16)
    return out.reshape(B, 1)
